# baseline (speedup 1.0000x reference)
import functools

import jax
import jax.numpy as jnp
from jax import lax
from jax.experimental import pallas as pl
from jax.experimental.pallas import tpu as pltpu
from jax.experimental.pallas import tpu_sc as plsc

_N = 10000
_E = 320000
_D = 128
_G = 64
_W3P = 16
_NC = 2
_NS = 16
_NW = _NC * _NS
_K = 80
_TPW = _E // _NW
_NCH = _TPW // _K
_RPS = 624
_RLAST = _N - _RPS * (_NS - 1)
_NP1 = 10240
_RPS1 = _NP1 // _NS

_mesh = plsc.VectorSubcoreMesh(core_axis_name="c", subcore_axis_name="s",
                               num_cores=_NC, num_subcores=_NS)


def _stripe_copy(src, dst, sid):
    @pl.when(sid < _NS - 1)
    def _():
        pltpu.sync_copy(src.at[pl.ds(sid * _RPS, _RPS)],
                        dst.at[pl.ds(sid * _RPS, _RPS)])

    @pl.when(sid == _NS - 1)
    def _():
        pltpu.sync_copy(src.at[pl.ds((_NS - 1) * _RPS, _RLAST)],
                        dst.at[pl.ds((_NS - 1) * _RPS, _RLAST)])


def _stripe_copy1(src, dst, sid):
    pltpu.sync_copy(src.at[pl.ds(sid * _RPS1, _RPS1)],
                    dst.at[pl.ds(sid * _RPS1, _RPS1)])


def _make_agg(width):
    @functools.partial(
        pl.kernel,
        out_type=jax.ShapeDtypeStruct((_NC, _N, width), jnp.float32),
        mesh=_mesh,
        scratch_types=[
            pltpu.VMEM((_NCH, _K), jnp.int32),
            pltpu.VMEM((_NCH, _K), jnp.int32),
            pltpu.VMEM((_K, width), jnp.float32),
            pltpu.VMEM_SHARED((_N, width), jnp.float32),
        ],
    )
    def agg(z_hbm, src_hbm, dst_hbm, zeros_hbm, out_hbm,
            src_v, dst_v, rows_v, acc_sh):
        cid = lax.axis_index("c")
        sid = lax.axis_index("s")
        wid = cid * _NS + sid
        pltpu.sync_copy(src_hbm.at[wid], src_v)
        pltpu.sync_copy(dst_hbm.at[wid], dst_v)
        _stripe_copy(zeros_hbm, acc_sh, sid)
        plsc.subcore_barrier()

        @pl.loop(0, _NCH)
        def body(c):
            pltpu.sync_copy(z_hbm.at[src_v.at[c]], rows_v)
            pltpu.sync_copy(rows_v, acc_sh.at[dst_v.at[c]], add=True)

        plsc.subcore_barrier()
        _stripe_copy(acc_sh, out_hbm.at[cid], sid)

    return agg


_agg128 = _make_agg(_D)


@functools.partial(
    pl.kernel,
    out_type=jax.ShapeDtypeStruct((_NC, _NP1), jnp.float32),
    mesh=_mesh,
    scratch_types=[
        pltpu.VMEM((_NCH, _K), jnp.int32),
        pltpu.VMEM((_K,), jnp.float32),
        pltpu.VMEM_SHARED((_NP1,), jnp.float32),
    ],
)
def _deg_kernel(dst_hbm, zeros_hbm, out_hbm, dst_v, ones_v, acc_sh):
    cid = lax.axis_index("c")
    sid = lax.axis_index("s")
    wid = cid * _NS + sid
    for i in range(_K // 16):
        ones_v[pl.ds(i * 16, 16)] = jnp.full((16,), 1.0, jnp.float32)
    pltpu.sync_copy(dst_hbm.at[wid], dst_v)
    _stripe_copy1(zeros_hbm, acc_sh, sid)
    plsc.subcore_barrier()

    def body(c, carry):
        pltpu.sync_copy(ones_v, acc_sh.at[dst_v.at[c]], add=True)
        return carry

    lax.fori_loop(0, _NCH, body, 0)
    plsc.subcore_barrier()
    _stripe_copy1(acc_sh, out_hbm.at[cid], sid)


_BLK = 2000


def _dot(a, b):
    return jnp.dot(a, b, preferred_element_type=jnp.float32)


def _dot_hi(a, b):
    return jnp.dot(a, b, preferred_element_type=jnp.float32,
                   precision=lax.Precision.HIGHEST)


def _tc1_body(deg0_ref, deg1_ref, x_ref, w_ref, dinv_ref, z_ref):
    deg = deg0_ref[...] + deg1_ref[...] + 1.0
    dinv = lax.rsqrt(deg)
    dinv_ref[...] = dinv
    z_ref[...] = _dot(x_ref[...], w_ref[...]) * dinv


def _tc1(deg0, deg1, x, w1):
    return pl.pallas_call(
        _tc1_body,
        grid=(_N // _BLK,),
        in_specs=[
            pl.BlockSpec((_BLK, 1), lambda i: (i, 0)),
            pl.BlockSpec((_BLK, 1), lambda i: (i, 0)),
            pl.BlockSpec((_BLK, _D), lambda i: (i, 0)),
            pl.BlockSpec((_D, _D), lambda i: (0, 0)),
        ],
        out_specs=[
            pl.BlockSpec((_BLK, 1), lambda i: (i, 0)),
            pl.BlockSpec((_BLK, _D), lambda i: (i, 0)),
        ],
        out_shape=[
            jax.ShapeDtypeStruct((_N, 1), jnp.float32),
            jax.ShapeDtypeStruct((_N, _D), jnp.float32),
        ],
    )(deg0, deg1, x, w1)


def _tc2_body(s0_ref, s1_ref, zp_ref, dinv_ref, b_ref, w_ref, out_ref):
    dinv = dinv_ref[...]
    h = s0_ref[...] + s1_ref[...] + zp_ref[...]
    h = jnp.maximum(h * dinv + b_ref[...][None, :], 0.0)
    out_ref[...] = _dot(h, w_ref[...]) * dinv


def _tc_mid(s0, s1, zp, dinv, b, w):
    return pl.pallas_call(
        _tc2_body,
        grid=(_N // _BLK,),
        in_specs=[
            pl.BlockSpec((_BLK, _D), lambda i: (i, 0)),
            pl.BlockSpec((_BLK, _D), lambda i: (i, 0)),
            pl.BlockSpec((_BLK, _D), lambda i: (i, 0)),
            pl.BlockSpec((_BLK, 1), lambda i: (i, 0)),
            pl.BlockSpec((_D,), lambda i: (0,)),
            pl.BlockSpec((_D, _D), lambda i: (0, 0)),
        ],
        out_specs=pl.BlockSpec((_BLK, _D), lambda i: (i, 0)),
        out_shape=jax.ShapeDtypeStruct((_N, _D), jnp.float32),
    )(s0, s1, zp, dinv, b, w)


def _tcf_body(s0_ref, s1_ref, zp_ref, dinv_ref, batch_ref, b3_ref,
              wlp_ref, blp_ref, out_ref):
    dinv = dinv_ref[...]
    h = (s0_ref[...] + s1_ref[...] + zp_ref[...]) * dinv + b3_ref[...][None, :]
    batch = batch_ref[...]
    seg = lax.broadcasted_iota(jnp.int32, (_G, _N), 0)
    onehot = jnp.where(batch[None, :] == seg, 1.0, 0.0).astype(jnp.float32)
    counts = jnp.sum(onehot, axis=1)
    sums = _dot_hi(onehot, h)
    pooled = sums / jnp.maximum(counts, 1.0)[:, None]
    out_ref[...] = _dot(pooled, wlp_ref[...]) + blp_ref[...][None, :]


def _tc_final(s0, s1, zp, dinv, batch, b3, wlp, blp):
    return pl.pallas_call(
        _tcf_body,
        in_specs=[
            pl.BlockSpec((_N, _D), lambda: (0, 0)),
            pl.BlockSpec((_N, _D), lambda: (0, 0)),
            pl.BlockSpec((_N, _D), lambda: (0, 0)),
            pl.BlockSpec((_N, 1), lambda: (0, 0)),
            pl.BlockSpec((_N,), lambda: (0,)),
            pl.BlockSpec((_D,), lambda: (0,)),
            pl.BlockSpec((_D, _W3P), lambda: (0, 0)),
            pl.BlockSpec((_W3P,), lambda: (0,)),
        ],
        out_specs=pl.BlockSpec((_G, _W3P), lambda: (0, 0)),
        out_shape=jax.ShapeDtypeStruct((_G, _W3P), jnp.float32),
    )(s0, s1, zp, dinv, batch, b3, wlp, blp)


def kernel(x, edge_index, batch, W1, b1, W2, b2, W3, b3, Wl, bl):
    src = edge_index[0].reshape(_NW, _NCH, _K)
    dst = edge_index[1].reshape(_NW, _NCH, _K)
    zeros_n = jnp.zeros((_NP1,), jnp.float32)
    zeros_nd = jnp.zeros((_N, _D), jnp.float32)
    wl_pad = jnp.pad(Wl, ((0, 0), (0, _W3P - Wl.shape[1])))
    bl_pad = jnp.pad(bl, (0, _W3P - bl.shape[0]))

    degp = _deg_kernel(dst, zeros_n)
    dinv, z1p = _tc1(degp[0, :_N][:, None], degp[1, :_N][:, None], x, W1)
    s1 = _agg128(z1p, src, dst, zeros_nd)
    z2p = _tc_mid(s1[0], s1[1], z1p, dinv, b1, W2)
    s2 = _agg128(z2p, src, dst, zeros_nd)
    z3p = _tc_mid(s2[0], s2[1], z2p, dinv, b2, W3)
    s3 = _agg128(z3p, src, dst, zeros_nd)
    out16 = _tc_final(s3[0], s3[1], z3p, dinv, batch, b3, wl_pad, bl_pad)
    return out16[:, :2]


# speedup vs baseline: 1.4637x; 1.4637x over previous
import functools

import jax
import jax.numpy as jnp
from jax import lax
from jax.experimental import pallas as pl
from jax.experimental.pallas import tpu as pltpu
from jax.experimental.pallas import tpu_sc as plsc

_N = 10000
_E = 320000
_D = 128
_G = 64
_W3P = 16
_NC = 2
_NS = 16
_NW = _NC * _NS
_K = 80
_TPW = _E // _NW
_NCH = _TPW // _K
_SB = 25
_NB = _NCH // _SB
_RPS = 624
_RLAST = _N - _RPS * (_NS - 1)
_NP1 = 10240
_RPS1 = _NP1 // _NS

_mesh = plsc.VectorSubcoreMesh(core_axis_name="c", subcore_axis_name="s",
                               num_cores=_NC, num_subcores=_NS)


def _stripe_copy(src, dst, sid):
    @pl.when(sid < _NS - 1)
    def _():
        pltpu.sync_copy(src.at[pl.ds(sid * _RPS, _RPS)],
                        dst.at[pl.ds(sid * _RPS, _RPS)])

    @pl.when(sid == _NS - 1)
    def _():
        pltpu.sync_copy(src.at[pl.ds((_NS - 1) * _RPS, _RLAST)],
                        dst.at[pl.ds((_NS - 1) * _RPS, _RLAST)])


def _stripe_copy1(src, dst, sid):
    pltpu.sync_copy(src.at[pl.ds(sid * _RPS1, _RPS1)],
                    dst.at[pl.ds(sid * _RPS1, _RPS1)])


def _make_agg(width):
    @functools.partial(
        pl.kernel,
        out_type=jax.ShapeDtypeStruct((_NC, _N, width), jnp.float32),
        mesh=_mesh,
        scratch_types=[
            pltpu.VMEM((_SB, _K), jnp.int32),
            pltpu.VMEM((_SB, _K), jnp.int32),
            pltpu.VMEM((2, _K, width), jnp.float32),
            pltpu.VMEM_SHARED((_N, width), jnp.float32),
            pltpu.SemaphoreType.DMA((2,)),
        ],
    )
    def agg(z_hbm, src_hbm, dst_hbm, zeros_hbm, out_hbm,
            src_v, dst_v, rows_v, acc_sh, sems):
        cid = lax.axis_index("c")
        sid = lax.axis_index("s")
        wid = cid * _NS + sid
        _stripe_copy(zeros_hbm, acc_sh, sid)
        plsc.subcore_barrier()

        def load(j, q):
            pltpu.async_copy(z_hbm.at[src_v.at[j]], rows_v.at[q], sems.at[q])

        def flush(j, q):
            pltpu.make_async_copy(z_hbm.at[src_v.at[j]], rows_v.at[q],
                                  sems.at[q]).wait()
            pltpu.sync_copy(rows_v.at[q], acc_sh.at[dst_v.at[j]], add=True)

        @pl.loop(0, _NB)
        def blk(b):
            pltpu.sync_copy(src_hbm.at[wid].at[b], src_v)
            pltpu.sync_copy(dst_hbm.at[wid].at[b], dst_v)
            load(0, 0)

            @pl.loop(1, _SB, step=2)
            def body(j):
                load(j, 1)
                flush(j - 1, 0)
                load(j + 1, 0)
                flush(j, 1)

            flush(_SB - 1, 0)

        plsc.subcore_barrier()
        _stripe_copy(acc_sh, out_hbm.at[cid], sid)

    return agg


_agg128 = _make_agg(_D)


@functools.partial(
    pl.kernel,
    out_type=jax.ShapeDtypeStruct((_NC, _NP1), jnp.float32),
    mesh=_mesh,
    scratch_types=[
        pltpu.VMEM((_NCH, _K), jnp.int32),
        pltpu.VMEM((_K,), jnp.float32),
        pltpu.VMEM_SHARED((_NP1,), jnp.float32),
    ],
)
def _deg_kernel(dst_hbm, zeros_hbm, out_hbm, dst_v, ones_v, acc_sh):
    cid = lax.axis_index("c")
    sid = lax.axis_index("s")
    wid = cid * _NS + sid
    for i in range(_K // 16):
        ones_v[pl.ds(i * 16, 16)] = jnp.full((16,), 1.0, jnp.float32)
    pltpu.sync_copy(dst_hbm.at[wid], dst_v)
    _stripe_copy1(zeros_hbm, acc_sh, sid)
    plsc.subcore_barrier()

    def body(c, carry):
        pltpu.sync_copy(ones_v, acc_sh.at[dst_v.at[c]], add=True)
        return carry

    lax.fori_loop(0, _NCH, body, 0)
    plsc.subcore_barrier()
    _stripe_copy1(acc_sh, out_hbm.at[cid], sid)


_BLK = 2000


def _dot(a, b):
    return jnp.dot(a, b, preferred_element_type=jnp.float32)


def _dot_hi(a, b):
    return jnp.dot(a, b, preferred_element_type=jnp.float32,
                   precision=lax.Precision.HIGHEST)


def _tc1_body(deg0_ref, deg1_ref, x_ref, w_ref, dinv_ref, z_ref):
    deg = deg0_ref[...] + deg1_ref[...] + 1.0
    dinv = lax.rsqrt(deg)
    dinv_ref[...] = dinv
    z_ref[...] = _dot(x_ref[...], w_ref[...]) * dinv


def _tc1(deg0, deg1, x, w1):
    return pl.pallas_call(
        _tc1_body,
        grid=(_N // _BLK,),
        in_specs=[
            pl.BlockSpec((_BLK, 1), lambda i: (i, 0)),
            pl.BlockSpec((_BLK, 1), lambda i: (i, 0)),
            pl.BlockSpec((_BLK, _D), lambda i: (i, 0)),
            pl.BlockSpec((_D, _D), lambda i: (0, 0)),
        ],
        out_specs=[
            pl.BlockSpec((_BLK, 1), lambda i: (i, 0)),
            pl.BlockSpec((_BLK, _D), lambda i: (i, 0)),
        ],
        out_shape=[
            jax.ShapeDtypeStruct((_N, 1), jnp.float32),
            jax.ShapeDtypeStruct((_N, _D), jnp.float32),
        ],
    )(deg0, deg1, x, w1)


def _tc2_body(s0_ref, s1_ref, zp_ref, dinv_ref, b_ref, w_ref, out_ref):
    dinv = dinv_ref[...]
    h = s0_ref[...] + s1_ref[...] + zp_ref[...]
    h = jnp.maximum(h * dinv + b_ref[...][None, :], 0.0)
    out_ref[...] = _dot(h, w_ref[...]) * dinv


def _tc_mid(s0, s1, zp, dinv, b, w):
    return pl.pallas_call(
        _tc2_body,
        grid=(_N // _BLK,),
        in_specs=[
            pl.BlockSpec((_BLK, _D), lambda i: (i, 0)),
            pl.BlockSpec((_BLK, _D), lambda i: (i, 0)),
            pl.BlockSpec((_BLK, _D), lambda i: (i, 0)),
            pl.BlockSpec((_BLK, 1), lambda i: (i, 0)),
            pl.BlockSpec((_D,), lambda i: (0,)),
            pl.BlockSpec((_D, _D), lambda i: (0, 0)),
        ],
        out_specs=pl.BlockSpec((_BLK, _D), lambda i: (i, 0)),
        out_shape=jax.ShapeDtypeStruct((_N, _D), jnp.float32),
    )(s0, s1, zp, dinv, b, w)


def _tcf_body(s0_ref, s1_ref, zp_ref, dinv_ref, batch_ref, b3_ref,
              wlp_ref, blp_ref, out_ref):
    dinv = dinv_ref[...]
    h = (s0_ref[...] + s1_ref[...] + zp_ref[...]) * dinv + b3_ref[...][None, :]
    batch = batch_ref[...]
    seg = lax.broadcasted_iota(jnp.int32, (_G, _N), 0)
    onehot = jnp.where(batch[None, :] == seg, 1.0, 0.0).astype(jnp.float32)
    counts = jnp.sum(onehot, axis=1)
    sums = _dot_hi(onehot, h)
    pooled = sums / jnp.maximum(counts, 1.0)[:, None]
    out_ref[...] = _dot(pooled, wlp_ref[...]) + blp_ref[...][None, :]


def _tc_final(s0, s1, zp, dinv, batch, b3, wlp, blp):
    return pl.pallas_call(
        _tcf_body,
        in_specs=[
            pl.BlockSpec((_N, _D), lambda: (0, 0)),
            pl.BlockSpec((_N, _D), lambda: (0, 0)),
            pl.BlockSpec((_N, _D), lambda: (0, 0)),
            pl.BlockSpec((_N, 1), lambda: (0, 0)),
            pl.BlockSpec((_N,), lambda: (0,)),
            pl.BlockSpec((_D,), lambda: (0,)),
            pl.BlockSpec((_D, _W3P), lambda: (0, 0)),
            pl.BlockSpec((_W3P,), lambda: (0,)),
        ],
        out_specs=pl.BlockSpec((_G, _W3P), lambda: (0, 0)),
        out_shape=jax.ShapeDtypeStruct((_G, _W3P), jnp.float32),
    )(s0, s1, zp, dinv, batch, b3, wlp, blp)


def kernel(x, edge_index, batch, W1, b1, W2, b2, W3, b3, Wl, bl):
    src = edge_index[0].reshape(_NW, _NB, _SB, _K)
    dst = edge_index[1].reshape(_NW, _NB, _SB, _K)
    zeros_n = jnp.zeros((_NP1,), jnp.float32)
    zeros_nd = jnp.zeros((_N, _D), jnp.float32)
    wl_pad = jnp.pad(Wl, ((0, 0), (0, _W3P - Wl.shape[1])))
    bl_pad = jnp.pad(bl, (0, _W3P - bl.shape[0]))

    degp = _deg_kernel(edge_index[1].reshape(_NW, _NCH, _K), zeros_n)
    dinv, z1p = _tc1(degp[0, :_N][:, None], degp[1, :_N][:, None], x, W1)
    s1 = _agg128(z1p, src, dst, zeros_nd)
    z2p = _tc_mid(s1[0], s1[1], z1p, dinv, b1, W2)
    s2 = _agg128(z2p, src, dst, zeros_nd)
    z3p = _tc_mid(s2[0], s2[1], z2p, dinv, b2, W3)
    s3 = _agg128(z3p, src, dst, zeros_nd)
    out16 = _tc_final(s3[0], s3[1], z3p, dinv, batch, b3, wl_pad, bl_pad)
    return out16[:, :2]
